# baseline (speedup 1.0000x reference)
import jax
import jax.numpy as jnp
from jax.experimental import pallas as pl

_B = 4096
_F = 100
_D = 128
_T = _F + 1
_BB = 128


def _body(x_ref, w_ref, b_ref, c_ref, o_ref):
    tok = x_ref[...][:, :, None] * w_ref[...][None] + b_ref[...][None]
    cls = jnp.broadcast_to(c_ref[...].reshape(1, 1, _D), (_BB, 1, _D))
    o_ref[...] = jnp.concatenate([cls, tok], axis=1)


def kernel(x, weight, bias, cls_token):
    return pl.pallas_call(
        _body,
        grid=(_B // _BB,),
        in_specs=[
            pl.BlockSpec((_BB, _F), lambda i: (i, 0)),
            pl.BlockSpec((_F, _D), lambda i: (0, 0)),
            pl.BlockSpec((_F, _D), lambda i: (0, 0)),
            pl.BlockSpec((1, 1, _D), lambda i: (0, 0, 0)),
        ],
        out_specs=pl.BlockSpec((_BB, _T, _D), lambda i: (i, 0, 0)),
        out_shape=jax.ShapeDtypeStruct((_B, _T, _D), jnp.float32),
    )(x, weight, bias, cls_token)
